# baseline (speedup 1.0000x reference)
import functools
import math

import jax
import jax.numpy as jnp
from jax import lax
from jax.experimental import pallas as pl
from jax.experimental.pallas import tpu as pltpu, tpu_sc as plsc

_C = 160
_G = _C // 16
_HID = 128
_HG = _HID // 16
_NW = 32


@functools.partial(jax.jit, static_argnums=(2, 3))
def _run(tree_flat, conn_flat, node_num, total_rows):
    num_chunks = total_rows // _C
    inv_s = 1.0 / math.sqrt(_HID)

    mesh = plsc.VectorSubcoreMesh(core_axis_name="c", subcore_axis_name="s")

    @functools.partial(
        pl.kernel,
        out_type=jax.ShapeDtypeStruct((total_rows, _HID), jnp.float32),
        mesh=mesh,
        scratch_types=[
            pltpu.VMEM((_C,), jnp.int32),
            pltpu.VMEM((_C,), jnp.int32),
            pltpu.VMEM((_C, _HID), jnp.float32),
            pltpu.VMEM((_C, _HID), jnp.float32),
            pltpu.VMEM((_C, _HID), jnp.float32),
            pltpu.VMEM((_C, _HID), jnp.float32),
            pltpu.SemaphoreType.DMA,
            pltpu.SemaphoreType.DMA,
        ],
    )
    def k(tree_hbm, conn_hbm, out_hbm, idx_a, idx_b, x_a, x_b, p_a, p_b,
          sem_a, sem_b):
        wid = lax.axis_index("s") * 2 + lax.axis_index("c")
        n_iter = (num_chunks - 1 - wid) // _NW + 1

        def start_loads(cid, idx_v, x_v, p_v, sem):
            base = cid * _C
            cps = [pltpu.async_copy(tree_hbm.at[pl.ds(base, _C)], x_v, sem)]
            pltpu.sync_copy(conn_hbm.at[pl.ds(base, _C)], idx_v)
            for j in range(_G):
                batch_base = ((base + j * 16) // node_num) * node_num
                flat_idx = idx_v[pl.ds(j * 16, 16)] + batch_base
                cps.append(pltpu.async_copy(
                    tree_hbm.at[flat_idx], p_v.at[pl.ds(j * 16, 16)], sem))
            return cps

        def compute_store(cid, x_v, p_v, sem):

            @plsc.parallel_loop(0, _C // 2, unroll=1)
            def _rowpair(rp):
                lane = lax.iota(jnp.int32, 16)
                xs = [[], []]
                ss = [[], []]
                accs = [None, None]
                for h in range(2):
                    r = 2 * rp + h
                    ms = []
                    for c in range(_HG):
                        xc = x_v[r, pl.ds(c * 16, 16)]
                        sc = p_v[r, pl.ds(c * 16, 16)] - xc
                        xs[h].append(xc)
                        ss[h].append(sc)
                        ms.append(sc * xc)
                    t0 = [ms[0] + ms[1], ms[2] + ms[3],
                          ms[4] + ms[5], ms[6] + ms[7]]
                    t1 = [t0[0] + t0[1], t0[2] + t0[3]]
                    accs[h] = t1[0] + t1[1]
                a0 = accs[0] + accs[0].at[lane ^ 8].get(mode="promise_in_bounds")
                a1 = accs[1] + accs[1].at[lane ^ 8].get(mode="promise_in_bounds")
                c01 = jnp.where(lane < 8, a0, a1)
                d = c01
                for kk in (4, 2, 1):
                    d = d + d.at[lane ^ kk].get(mode="promise_in_bounds")
                w = 1.0 / (1.0 + jnp.exp(d * (-inv_s)))
                w0 = w.at[lane & 0].get(mode="promise_in_bounds")
                w1 = w.at[lane | 8].get(mode="promise_in_bounds")
                for c in range(_HG):
                    p_v[2 * rp, pl.ds(c * 16, 16)] = xs[0][c] + w0 * ss[0][c]
                    p_v[2 * rp + 1, pl.ds(c * 16, 16)] = xs[1][c] + w1 * ss[1][c]

            return pltpu.async_copy(p_v, out_hbm.at[pl.ds(cid * _C, _C)], sem)

        def pair_body(i, carry):
            c0 = wid + (2 * i) * _NW
            c1 = wid + (2 * i + 1) * _NW
            l0 = start_loads(c0, idx_a, x_a, p_a, sem_a)
            l1 = start_loads(c1, idx_b, x_b, p_b, sem_b)
            for cp in l0:
                cp.wait()
            o0 = compute_store(c0, x_a, p_a, sem_a)
            for cp in l1:
                cp.wait()
            o1 = compute_store(c1, x_b, p_b, sem_b)
            o0.wait()
            o1.wait()
            return carry

        lax.fori_loop(0, n_iter // 2, pair_body, 0)

        @pl.when(n_iter % 2 == 1)
        def _tail():
            cid = wid + (n_iter - 1) * _NW
            for cp in start_loads(cid, idx_a, x_a, p_a, sem_a):
                cp.wait()
            compute_store(cid, x_a, p_a, sem_a).wait()

    return k(tree_flat, conn_flat)


def kernel(tree_embedding, node_connection, node_mask):
    batch, node_num, hid = tree_embedding.shape
    assert hid == _HID and node_num % 16 == 0
    total_rows = batch * node_num
    assert total_rows % _C == 0
    tree_flat = tree_embedding.reshape(total_rows, hid)
    conn_flat = node_connection.astype(jnp.int32).reshape(total_rows)
    out = _run(tree_flat, conn_flat, node_num, total_rows)
    return out.reshape(batch, node_num, hid)


# speedup vs baseline: 1.2636x; 1.2636x over previous
import functools
import math

import jax
import jax.numpy as jnp
from jax import lax
from jax.experimental import pallas as pl
from jax.experimental.pallas import tpu as pltpu, tpu_sc as plsc

_C = 160
_G = _C // 16
_HID = 128
_HG = _HID // 16
_NW = 32


@functools.partial(jax.jit, static_argnums=(2, 3))
def _run(tree_flat, conn_flat, node_num, total_rows):
    num_chunks = total_rows // _C
    inv_s = 1.0 / math.sqrt(_HID)

    mesh = plsc.VectorSubcoreMesh(core_axis_name="c", subcore_axis_name="s")

    @functools.partial(
        pl.kernel,
        out_type=jax.ShapeDtypeStruct((total_rows, _HID), jnp.float32),
        mesh=mesh,
        scratch_types=[
            pltpu.VMEM((_C,), jnp.int32),
            pltpu.VMEM((_C,), jnp.int32),
            pltpu.VMEM((_C, _HID), jnp.float32),
            pltpu.VMEM((_C, _HID), jnp.float32),
            pltpu.VMEM((_C, _HID), jnp.float32),
            pltpu.VMEM((_C, _HID), jnp.float32),
            pltpu.SemaphoreType.DMA,
            pltpu.SemaphoreType.DMA,
            pltpu.SemaphoreType.DMA,
            pltpu.SemaphoreType.DMA,
        ],
    )
    def k(tree_hbm, conn_hbm, out_hbm, idx_a, idx_b, x_a, x_b, p_a, p_b,
          sem_a, sem_b, semo_a, semo_b):
        wid = lax.axis_index("s") * 2 + lax.axis_index("c")
        n_iter = (num_chunks - 1 - wid) // _NW + 1

        def start_loads(cid, idx_v, x_v, p_v, sem):
            base = cid * _C
            cps = [pltpu.async_copy(tree_hbm.at[pl.ds(base, _C)], x_v, sem)]
            pltpu.sync_copy(conn_hbm.at[pl.ds(base, _C)], idx_v)
            for j in range(_G):
                batch_base = ((base + j * 16) // node_num) * node_num
                flat_idx = idx_v[pl.ds(j * 16, 16)] + batch_base
                cps.append(pltpu.async_copy(
                    tree_hbm.at[flat_idx], p_v.at[pl.ds(j * 16, 16)], sem))
            return cps

        def compute_store(cid, x_v, p_v, sem):

            @plsc.parallel_loop(0, _C, unroll=1)
            def _row(r):
                xs = []
                ss = []
                ms = []
                for c in range(_HG):
                    xc = x_v[r, pl.ds(c * 16, 16)]
                    sc = p_v[r, pl.ds(c * 16, 16)] - xc
                    xs.append(xc)
                    ss.append(sc)
                    ms.append(sc * xc)
                t0 = [ms[0] + ms[1], ms[2] + ms[3], ms[4] + ms[5], ms[6] + ms[7]]
                t1 = [t0[0] + t0[1], t0[2] + t0[3]]
                acc = t1[0] + t1[1]
                lane = lax.iota(jnp.int32, 16)
                d = acc
                for kk in (8, 4, 2, 1):
                    d = d + d.at[lane ^ kk].get(mode="promise_in_bounds")
                w = 1.0 / (1.0 + jnp.exp(d * (-inv_s)))
                for c in range(_HG):
                    p_v[r, pl.ds(c * 16, 16)] = xs[c] + w * ss[c]

            return pltpu.async_copy(p_v, out_hbm.at[pl.ds(cid * _C, _C)], sem)

        def drain_outs():
            pltpu.make_async_copy(p_a, out_hbm.at[pl.ds(0, _C)], semo_a).wait()
            pltpu.make_async_copy(p_b, out_hbm.at[pl.ds(0, _C)], semo_b).wait()

        def pair_body(i, carry):
            c0 = wid + (2 * i) * _NW
            c1 = wid + (2 * i + 1) * _NW
            @pl.when(i > 0)
            def _drain():
                drain_outs()

            l0 = start_loads(c0, idx_a, x_a, p_a, sem_a)
            l1 = start_loads(c1, idx_b, x_b, p_b, sem_b)
            for cp in l0:
                cp.wait()
            compute_store(c0, x_a, p_a, semo_a)
            for cp in l1:
                cp.wait()
            compute_store(c1, x_b, p_b, semo_b)
            return carry

        lax.fori_loop(0, n_iter // 2, pair_body, 0)
        drain_outs()

        @pl.when(n_iter % 2 == 1)
        def _tail():
            cid = wid + (n_iter - 1) * _NW
            for cp in start_loads(cid, idx_a, x_a, p_a, sem_a):
                cp.wait()
            compute_store(cid, x_a, p_a, semo_a).wait()

    return k(tree_flat, conn_flat)


def kernel(tree_embedding, node_connection, node_mask):
    batch, node_num, hid = tree_embedding.shape
    assert hid == _HID and node_num % 16 == 0
    total_rows = batch * node_num
    assert total_rows % _C == 0
    tree_flat = tree_embedding.reshape(total_rows, hid)
    conn_flat = node_connection.astype(jnp.int32).reshape(total_rows)
    out = _run(tree_flat, conn_flat, node_num, total_rows)
    return out.reshape(batch, node_num, hid)


# speedup vs baseline: 1.4607x; 1.1560x over previous
import functools
import math

import jax
import jax.numpy as jnp
from jax import lax
from jax.experimental import pallas as pl
from jax.experimental.pallas import tpu as pltpu, tpu_sc as plsc

_C = 160
_G = _C // 16
_HID = 128
_HG = _HID // 16
_NW = 32


@functools.partial(jax.jit, static_argnums=(2, 3))
def _run(tree_flat, conn_flat, node_num, total_rows):
    num_chunks = total_rows // _C
    inv_s = 1.0 / math.sqrt(_HID)

    mesh = plsc.VectorSubcoreMesh(core_axis_name="c", subcore_axis_name="s")

    @functools.partial(
        pl.kernel,
        out_type=jax.ShapeDtypeStruct((total_rows, _HID), jnp.float32),
        mesh=mesh,
        scratch_types=[
            pltpu.VMEM((_C,), jnp.int32),
            pltpu.VMEM((_C,), jnp.int32),
            pltpu.VMEM((_C, _HID), jnp.float32),
            pltpu.VMEM((_C, _HID), jnp.float32),
            pltpu.VMEM((_C, _HID), jnp.float32),
            pltpu.VMEM((_C, _HID), jnp.float32),
            pltpu.SemaphoreType.DMA,
            pltpu.SemaphoreType.DMA,
            pltpu.SemaphoreType.DMA,
            pltpu.SemaphoreType.DMA,
        ],
    )
    def k(tree_hbm, conn_hbm, out_hbm, idx_a, idx_b, x_a, x_b, p_a, p_b,
          sem_a, sem_b, semo_a, semo_b):
        wid = lax.axis_index("s") * 2 + lax.axis_index("c")
        n_iter = (num_chunks - 1 - wid) // _NW + 1
        buf_a = (idx_a, x_a, p_a, sem_a, semo_a)
        buf_b = (idx_b, x_b, p_b, sem_b, semo_b)

        def fire_loads(buf, cid):
            idx_v, x_v, p_v, sem, _ = buf
            base = cid * _C
            pltpu.async_copy(tree_hbm.at[pl.ds(base, _C)], x_v, sem)
            pltpu.sync_copy(conn_hbm.at[pl.ds(base, _C)], idx_v)
            for j in range(_G):
                batch_base = ((base + j * 16) // node_num) * node_num
                flat_idx = idx_v[pl.ds(j * 16, 16)] + batch_base
                pltpu.async_copy(
                    tree_hbm.at[flat_idx], p_v.at[pl.ds(j * 16, 16)], sem)

        def drain_loads(buf):
            idx_v, x_v, p_v, sem, _ = buf
            pltpu.make_async_copy(tree_hbm.at[pl.ds(0, _C)], x_v, sem).wait()
            zeros = jnp.zeros((16,), jnp.int32)
            for j in range(_G):
                pltpu.make_async_copy(
                    tree_hbm.at[zeros], p_v.at[pl.ds(j * 16, 16)], sem).wait()

        def drain_out(buf):
            _, _, p_v, _, semo = buf
            pltpu.make_async_copy(p_v, out_hbm.at[pl.ds(0, _C)], semo).wait()

        def compute_store(buf, cid):
            _, x_v, p_v, _, semo = buf

            @plsc.parallel_loop(0, _C, unroll=1)
            def _row(r):
                xs = []
                ss = []
                ms = []
                for c in range(_HG):
                    xc = x_v[r, pl.ds(c * 16, 16)]
                    sc = p_v[r, pl.ds(c * 16, 16)] - xc
                    xs.append(xc)
                    ss.append(sc)
                    ms.append(sc * xc)
                t0 = [ms[0] + ms[1], ms[2] + ms[3], ms[4] + ms[5], ms[6] + ms[7]]
                t1 = [t0[0] + t0[1], t0[2] + t0[3]]
                acc = t1[0] + t1[1]
                lane = lax.iota(jnp.int32, 16)
                d = acc
                for kk in (8, 4, 2, 1):
                    d = d + d.at[lane ^ kk].get(mode="promise_in_bounds")
                w = 1.0 / (1.0 + jnp.exp(d * (-inv_s)))
                for c in range(_HG):
                    p_v[r, pl.ds(c * 16, 16)] = xs[c] + w * ss[c]

            pltpu.async_copy(p_v, out_hbm.at[pl.ds(cid * _C, _C)], semo)

        def do_step(cur, nxt, t):
            cid = wid + t * _NW
            @pl.when(t >= 1)
            def _do():
                drain_out(nxt)

            @pl.when(t + 1 < n_iter)
            def _pf():
                fire_loads(nxt, cid + _NW)

            drain_loads(cur)
            compute_store(cur, cid)

        def body(t, carry):
            @pl.when(t % 2 == 0)
            def _even():
                do_step(buf_a, buf_b, t)

            @pl.when(t % 2 == 1)
            def _odd():
                do_step(buf_b, buf_a, t)

            return carry

        fire_loads(buf_a, wid)
        lax.fori_loop(0, n_iter, body, 0)

        @pl.when((n_iter - 1) % 2 == 0)
        def _fin_a():
            drain_out(buf_a)

        @pl.when((n_iter - 1) % 2 == 1)
        def _fin_b():
            drain_out(buf_b)

    return k(tree_flat, conn_flat)


def kernel(tree_embedding, node_connection, node_mask):
    batch, node_num, hid = tree_embedding.shape
    assert hid == _HID and node_num % 16 == 0
    total_rows = batch * node_num
    assert total_rows % _C == 0
    tree_flat = tree_embedding.reshape(total_rows, hid)
    conn_flat = node_connection.astype(jnp.int32).reshape(total_rows)
    out = _run(tree_flat, conn_flat, node_num, total_rows)
    return out.reshape(batch, node_num, hid)
